# Initial kernel scaffold; baseline (speedup 1.0000x reference)
import jax
import jax.numpy as jnp
from jax.experimental import pallas as pl


def kernel(sim_data, train_data, Em_table, Ed_table, W1, b1, W2, b2):
    raise NotImplementedError("write your pallas kernel here")


# speedup vs baseline: 3.5228x; 3.5228x over previous
import functools

import jax
import jax.numpy as jnp
from jax import lax
from jax.experimental import pallas as pl
from jax.experimental.pallas import tpu as pltpu
from jax.experimental.pallas import tpu_sc as plsc

NUM_EMB = 100000
EMB_DIM = 64
BATCH = 16384
HIDDEN = 64

NC = 2
NS = 16
NW = NC * NS
IDX_W = 128
N_ROWS = BATCH // IDX_W
ROWS_PER_W = N_ROWS // NW


def _sc_gather(sim_data, m_idx, d_idx, Em_table, Ed_table):
    mesh = plsc.VectorSubcoreMesh(core_axis_name="c", subcore_axis_name="s")

    @functools.partial(
        pl.kernel,
        mesh=mesh,
        out_type=[
            jax.ShapeDtypeStruct((N_ROWS, IDX_W, EMB_DIM), jnp.float32),
            jax.ShapeDtypeStruct((N_ROWS, IDX_W, EMB_DIM), jnp.float32),
        ],
        scratch_types=[
            pltpu.VMEM((ROWS_PER_W, IDX_W), jnp.int32),
            pltpu.VMEM((ROWS_PER_W, IDX_W), jnp.int32),
            pltpu.VMEM((ROWS_PER_W, IDX_W), jnp.int32),
            pltpu.VMEM((ROWS_PER_W, IDX_W), jnp.int32),
            pltpu.VMEM((ROWS_PER_W, IDX_W, EMB_DIM), jnp.float32),
            pltpu.VMEM((ROWS_PER_W, IDX_W, EMB_DIM), jnp.float32),
            pltpu.SemaphoreType.DMA,
        ],
        compiler_params=pltpu.CompilerParams(use_tc_tiling_on_sc=False),
    )
    def gather_kernel(sim_hbm, midx_hbm, didx_hbm, em_hbm, ed_hbm,
                      outm_hbm, outd_hbm,
                      mi_v, di_v, sm_v, sd_v, mrows_v, drows_v, sem):
        wid = lax.axis_index("s") * NC + lax.axis_index("c")
        rbase = wid * ROWS_PER_W
        pltpu.sync_copy(midx_hbm.at[pl.ds(rbase, ROWS_PER_W)], mi_v)
        pltpu.sync_copy(didx_hbm.at[pl.ds(rbase, ROWS_PER_W)], di_v)
        for j in range(ROWS_PER_W):
            pltpu.async_copy(sim_hbm.at[mi_v.at[j]], sm_v.at[j], sem).wait()
            pltpu.async_copy(sim_hbm.at[di_v.at[j]], sd_v.at[j], sem).wait()
            pltpu.async_copy(em_hbm.at[sm_v.at[j]], mrows_v.at[j], sem).wait()
            pltpu.async_copy(ed_hbm.at[sd_v.at[j]], drows_v.at[j], sem).wait()
        pltpu.sync_copy(mrows_v, outm_hbm.at[pl.ds(rbase, ROWS_PER_W)])
        pltpu.sync_copy(drows_v, outd_hbm.at[pl.ds(rbase, ROWS_PER_W)])

    return gather_kernel(sim_data, m_idx, d_idx, Em_table, Ed_table)


def _mlp_body(m_ref, d_ref, w1_ref, b1_ref, w2_ref, b2_ref, out_ref):
    h = jnp.dot(m_ref[...], w1_ref[0:EMB_DIM, :],
                preferred_element_type=jnp.float32)
    h = h + jnp.dot(d_ref[...], w1_ref[EMB_DIM:2 * EMB_DIM, :],
                    preferred_element_type=jnp.float32)
    h = jax.nn.relu(h + b1_ref[...])
    z = jnp.dot(h, w2_ref[...], preferred_element_type=jnp.float32)
    out_ref[...] = jax.nn.sigmoid(z + b2_ref[...])


def _tc_mlp(mRows, dRows, W1, b1, W2, b2):
    blk = 2048
    grid = (BATCH // blk,)
    return pl.pallas_call(
        _mlp_body,
        grid=grid,
        in_specs=[
            pl.BlockSpec((blk, EMB_DIM), lambda i: (i, 0)),
            pl.BlockSpec((blk, EMB_DIM), lambda i: (i, 0)),
            pl.BlockSpec((2 * EMB_DIM, HIDDEN), lambda i: (0, 0)),
            pl.BlockSpec((1, HIDDEN), lambda i: (0, 0)),
            pl.BlockSpec((HIDDEN, 1), lambda i: (0, 0)),
            pl.BlockSpec((1, 1), lambda i: (0, 0)),
        ],
        out_specs=pl.BlockSpec((blk, 1), lambda i: (i, 0)),
        out_shape=jax.ShapeDtypeStruct((BATCH, 1), jnp.float32),
    )(mRows, dRows, W1, b1, W2, b2)


def kernel(sim_data, train_data, Em_table, Ed_table, W1, b1, W2, b2):
    m_idx = train_data[:, 0].reshape(N_ROWS, IDX_W)
    d_idx = train_data[:, 1].reshape(N_ROWS, IDX_W)
    mRows, dRows = _sc_gather(sim_data, m_idx, d_idx, Em_table, Ed_table)
    mRows = mRows.reshape(BATCH, EMB_DIM)
    dRows = dRows.reshape(BATCH, EMB_DIM)
    out = _tc_mlp(mRows, dRows, W1, b1.reshape(1, HIDDEN), W2,
                  b2.reshape(1, 1))
    return out.reshape(BATCH)


# speedup vs baseline: 4.4311x; 1.2578x over previous
import functools

import jax
import jax.numpy as jnp
from jax import lax
from jax.experimental import pallas as pl
from jax.experimental.pallas import tpu as pltpu
from jax.experimental.pallas import tpu_sc as plsc

NUM_EMB = 100000
EMB_DIM = 64
BATCH = 16384
HIDDEN = 64

NC = 2
NS = 16
NW = NC * NS
IDX_W = 128
ROWS_PER_W = BATCH // (NW * IDX_W)


def _sc_compose(sim_data, m_idx, d_idx):
    mesh = plsc.VectorSubcoreMesh(core_axis_name="c", subcore_axis_name="s")

    @functools.partial(
        pl.kernel,
        mesh=mesh,
        out_type=[
            jax.ShapeDtypeStruct((NW, ROWS_PER_W, IDX_W), jnp.int32),
            jax.ShapeDtypeStruct((NW, ROWS_PER_W, IDX_W), jnp.int32),
        ],
        scratch_types=[
            pltpu.VMEM((ROWS_PER_W, IDX_W), jnp.int32),
            pltpu.VMEM((ROWS_PER_W, IDX_W), jnp.int32),
            pltpu.VMEM((ROWS_PER_W, IDX_W), jnp.int32),
            pltpu.VMEM((ROWS_PER_W, IDX_W), jnp.int32),
            pltpu.SemaphoreType.DMA,
        ],
        compiler_params=pltpu.CompilerParams(use_tc_tiling_on_sc=False),
    )
    def compose_kernel(sim_hbm, midx_hbm, didx_hbm, outm_hbm, outd_hbm,
                       mi_v, di_v, sm_v, sd_v, sem):
        wid = lax.axis_index("s") * NC + lax.axis_index("c")
        pltpu.sync_copy(midx_hbm.at[wid], mi_v)
        pltpu.sync_copy(didx_hbm.at[wid], di_v)
        copies = []
        for j in range(ROWS_PER_W):
            copies.append(
                pltpu.async_copy(sim_hbm.at[mi_v.at[j]], sm_v.at[j], sem))
            copies.append(
                pltpu.async_copy(sim_hbm.at[di_v.at[j]], sd_v.at[j], sem))
        for c in copies:
            c.wait()
        pltpu.sync_copy(sm_v, outm_hbm.at[wid])
        pltpu.sync_copy(sd_v, outd_hbm.at[wid])

    return compose_kernel(sim_data, m_idx, d_idx)


def _sc_row_gather(table, sm, sd):
    mesh = plsc.VectorSubcoreMesh(core_axis_name="c", subcore_axis_name="s")
    out_sh = jax.ShapeDtypeStruct(
        (NW, ROWS_PER_W, IDX_W, 2 * EMB_DIM), jnp.float32)

    @functools.partial(
        pl.kernel,
        mesh=mesh,
        out_type=[out_sh, out_sh],
        scratch_types=[
            pltpu.VMEM((ROWS_PER_W, IDX_W), jnp.int32),
            pltpu.VMEM((ROWS_PER_W, IDX_W), jnp.int32),
            pltpu.VMEM((ROWS_PER_W, IDX_W, 2 * EMB_DIM), jnp.float32),
            pltpu.SemaphoreType.DMA,
        ],
    )
    def gather_kernel(table_hbm, sm_hbm, sd_hbm, outm_hbm, outd_hbm,
                      sm_v, sd_v, rows_v, sem):
        wid = lax.axis_index("s") * NC + lax.axis_index("c")
        pltpu.sync_copy(sm_hbm.at[wid], sm_v)
        pltpu.sync_copy(sd_hbm.at[wid], sd_v)
        copies = [pltpu.async_copy(table_hbm.at[sm_v.at[j]], rows_v.at[j], sem)
                  for j in range(ROWS_PER_W)]
        for c in copies:
            c.wait()
        pltpu.sync_copy(rows_v, outm_hbm.at[wid])
        copies = [pltpu.async_copy(table_hbm.at[sd_v.at[j]], rows_v.at[j], sem)
                  for j in range(ROWS_PER_W)]
        for c in copies:
            c.wait()
        pltpu.sync_copy(rows_v, outd_hbm.at[wid])

    return gather_kernel(table, sm, sd)


def _mlp_body(m_ref, d_ref, w1m_ref, w1d_ref, b1_ref, w2_ref, b2_ref,
              out_ref):
    h = jnp.dot(m_ref[...], w1m_ref[...], preferred_element_type=jnp.float32)
    h = h + jnp.dot(d_ref[...], w1d_ref[...],
                    preferred_element_type=jnp.float32)
    h = jax.nn.relu(h + b1_ref[...])
    z = jnp.dot(h, w2_ref[...], preferred_element_type=jnp.float32)
    out_ref[...] = jax.nn.sigmoid(z + b2_ref[...])


def _tc_mlp(gm, gd, W1m, W1d, b1, W2, b2):
    blk = 2048
    grid = (BATCH // blk,)
    return pl.pallas_call(
        _mlp_body,
        grid=grid,
        in_specs=[
            pl.BlockSpec((blk, 2 * EMB_DIM), lambda i: (i, 0)),
            pl.BlockSpec((blk, 2 * EMB_DIM), lambda i: (i, 0)),
            pl.BlockSpec((2 * EMB_DIM, HIDDEN), lambda i: (0, 0)),
            pl.BlockSpec((2 * EMB_DIM, HIDDEN), lambda i: (0, 0)),
            pl.BlockSpec((1, HIDDEN), lambda i: (0, 0)),
            pl.BlockSpec((HIDDEN, 1), lambda i: (0, 0)),
            pl.BlockSpec((1, 1), lambda i: (0, 0)),
        ],
        out_specs=pl.BlockSpec((blk, 1), lambda i: (i, 0)),
        out_shape=jax.ShapeDtypeStruct((BATCH, 1), jnp.float32),
    )(gm, gd, W1m, W1d, b1, W2, b2)


def kernel(sim_data, train_data, Em_table, Ed_table, W1, b1, W2, b2):
    m_idx = train_data[:, 0].reshape(NW, ROWS_PER_W, IDX_W)
    d_idx = train_data[:, 1].reshape(NW, ROWS_PER_W, IDX_W)
    table = jnp.concatenate([Em_table, Ed_table], axis=1)
    sm, sd = _sc_compose(sim_data, m_idx, d_idx)
    gm, gd = _sc_row_gather(table, sm, sd)
    gm = gm.reshape(BATCH, 2 * EMB_DIM)
    gd = gd.reshape(BATCH, 2 * EMB_DIM)
    zeros = jnp.zeros((EMB_DIM, HIDDEN), jnp.float32)
    W1m = jnp.concatenate([W1[:EMB_DIM], zeros], axis=0)
    W1d = jnp.concatenate([zeros, W1[EMB_DIM:]], axis=0)
    out = _tc_mlp(gm, gd, W1m, W1d, b1.reshape(1, HIDDEN), W2,
                  b2.reshape(1, 1))
    return out.reshape(BATCH)
